# baseline (speedup 1.0000x reference)
import functools

import jax
import jax.numpy as jnp
from jax.experimental import pallas as pl
from jax.experimental.pallas import tpu as pltpu

_BLK = 512
_NSTREAM = 2


def _gate_block(xa_ref, xb_ref, wt_ref, b_ref, oa_ref, ob_ref):
    for x_ref, o_ref in ((xa_ref, oa_ref), (xb_ref, ob_ref)):
        logits = jnp.dot(x_ref[...], wt_ref[...],
                         preferred_element_type=jnp.float32) + b_ref[...]
        l0 = logits[:, 0:1]
        lrest = jnp.max(logits[:, 1:], axis=1, keepdims=True)
        is0 = (l0 >= lrest).astype(jnp.float32)
        o_ref[...] = jnp.concatenate([is0, 1.0 - is0], axis=1)


@jax.jit
def kernel(x, W, b):
    B, S, D = x.shape
    K = W.shape[0]
    M = B * S
    H = M // _NSTREAM
    x2 = x.reshape(M, D)
    wt = W.T
    b2 = b.reshape(1, K)
    xa, xb = x2[:H], x2[H:]
    oa, ob = pl.pallas_call(
        _gate_block,
        grid=(H // _BLK,),
        in_specs=[
            pl.BlockSpec((_BLK, D), lambda i: (i, 0)),
            pl.BlockSpec((_BLK, D), lambda i: (i, 0)),
            pl.BlockSpec((D, K), lambda i: (0, 0)),
            pl.BlockSpec((1, K), lambda i: (0, 0)),
        ],
        out_specs=[
            pl.BlockSpec((_BLK, 2), lambda i: (i, 0)),
            pl.BlockSpec((_BLK, 2), lambda i: (i, 0)),
        ],
        out_shape=[
            jax.ShapeDtypeStruct((H, 2), jnp.float32),
            jax.ShapeDtypeStruct((H, 2), jnp.float32),
        ],
        compiler_params=pltpu.CompilerParams(
            dimension_semantics=("arbitrary",),
        ),
    )(xa, xb, wt, b2)
    return jnp.concatenate([oa, ob], axis=0).reshape(B, S, 2)


# speedup vs baseline: 2.7635x; 2.7635x over previous
import functools

import jax
import jax.numpy as jnp
from jax.experimental import pallas as pl
from jax.experimental.pallas import tpu as pltpu

_BLK = 512
_NSTREAM = 2


def _gate_block(xa_ref, xb_ref, wt_ref, b_ref, oa_ref, ob_ref):
    for x_ref, o_ref in ((xa_ref, oa_ref), (xb_ref, ob_ref)):
        logits = jnp.dot(x_ref[...], wt_ref[...],
                         preferred_element_type=jnp.float32) + b_ref[...]
        l0 = logits[:, 0:1]
        lrest = jnp.max(logits[:, 1:], axis=1, keepdims=True)
        is0 = (l0 >= lrest).astype(jnp.float32)
        o_ref[...] = jnp.concatenate([is0, 1.0 - is0], axis=1)


@jax.jit
def kernel(x, W, b):
    B, S, D = x.shape
    K = W.shape[0]
    M = B * S
    H = M // _NSTREAM
    x2 = x.reshape(M, D)
    wt = W.T
    b2 = b.reshape(1, K)
    nb = H // _BLK
    oa, ob = pl.pallas_call(
        _gate_block,
        grid=(nb,),
        in_specs=[
            pl.BlockSpec((_BLK, D), lambda i: (i, 0)),
            pl.BlockSpec((_BLK, D), lambda i: (i + nb, 0)),
            pl.BlockSpec((D, K), lambda i: (0, 0)),
            pl.BlockSpec((1, K), lambda i: (0, 0)),
        ],
        out_specs=[
            pl.BlockSpec((_BLK, 2), lambda i: (i, 0)),
            pl.BlockSpec((_BLK, 2), lambda i: (i, 0)),
        ],
        out_shape=[
            jax.ShapeDtypeStruct((H, 2), jnp.float32),
            jax.ShapeDtypeStruct((H, 2), jnp.float32),
        ],
        compiler_params=pltpu.CompilerParams(
            dimension_semantics=("arbitrary",),
        ),
    )(x2, x2, wt, b2)
    return jnp.concatenate([oa, ob], axis=0).reshape(B, S, 2)
